# Initial kernel scaffold; baseline (speedup 1.0000x reference)
import jax
import jax.numpy as jnp
from jax.experimental import pallas as pl


def kernel(x, batch, data, W, b):
    raise NotImplementedError("write your pallas kernel here")


# speedup vs baseline: 4.2700x; 4.2700x over previous
import functools

import jax
import jax.numpy as jnp
from jax import lax
from jax.experimental import pallas as pl
from jax.experimental.pallas import tpu as pltpu
from jax.experimental.pallas import tpu_sc as plsc

N = 320000
D = 128
S = 4096

NC = 2
NS = 16
NW = NC * NS

ROWS_PER_W = N // NW
CHUNK = 400
SUB = 80
NSUB = CHUNK // SUB
NCHUNK = ROWS_PER_W // CHUNK
SEG_PER_TILE = S // NS

_MESH = plsc.VectorSubcoreMesh(core_axis_name="c", subcore_axis_name="s")


def _sc_segment_sum(x, batch):
    @functools.partial(
        pl.kernel,
        out_type=jax.ShapeDtypeStruct((NC, S, D), jnp.float32),
        mesh=_MESH,
        scratch_types=[
            pltpu.VMEM((CHUNK, D), jnp.float32),
            pltpu.VMEM((NSUB, SUB), jnp.int32),
            pltpu.VMEM_SHARED((S, D), jnp.float32),
        ],
    )
    def k(x_hbm, b_hbm, sums_hbm, xbuf, idxbuf, acc):
        cid = lax.axis_index("c")
        sid = lax.axis_index("s")
        wid = cid * NS + sid

        def zrow(i, _):
            for j in range(D // 16):
                xbuf[i, pl.ds(j * 16, 16)] = jnp.zeros((16,), jnp.float32)
            return 0

        lax.fori_loop(0, SEG_PER_TILE, zrow, 0)
        seg0 = sid * SEG_PER_TILE
        pltpu.sync_copy(xbuf.at[pl.ds(0, SEG_PER_TILE)],
                        acc.at[pl.ds(seg0, SEG_PER_TILE)])
        plsc.subcore_barrier()

        base = wid * ROWS_PER_W

        def chunk_body(kk, _):
            row0 = base + kk * CHUNK
            pltpu.sync_copy(x_hbm.at[pl.ds(row0, CHUNK)], xbuf)
            for j in range(NSUB):
                pltpu.sync_copy(b_hbm.at[pl.ds(row0 + j * SUB, SUB)],
                                idxbuf.at[j])
            for j in range(NSUB):
                pltpu.sync_copy(xbuf.at[pl.ds(j * SUB, SUB)],
                                acc.at[idxbuf.at[j]], add=True)
            return 0

        lax.fori_loop(0, NCHUNK, chunk_body, 0)

        plsc.subcore_barrier()
        pltpu.sync_copy(acc.at[pl.ds(seg0, SEG_PER_TILE)],
                        sums_hbm.at[cid, pl.ds(seg0, SEG_PER_TILE)])

    return k(x, batch)


def _sc_segment_count(batch):
    @functools.partial(
        pl.kernel,
        out_type=jax.ShapeDtypeStruct((NC * S,), jnp.float32),
        mesh=_MESH,
        scratch_types=[
            pltpu.VMEM((NSUB, SUB), jnp.int32),
            pltpu.VMEM((SUB, D), jnp.float32),
            pltpu.VMEM((SEG_PER_TILE, D), jnp.float32),
            pltpu.VMEM((SEG_PER_TILE,), jnp.float32),
            pltpu.VMEM_SHARED((S, D), jnp.float32),
        ],
    )
    def k(b_hbm, cnts_hbm, idxbuf, ones, zc, c1d, cacc):
        cid = lax.axis_index("c")
        sid = lax.axis_index("s")
        wid = cid * NS + sid

        def zrow(i, _):
            for j in range(D // 16):
                zc[i, pl.ds(j * 16, 16)] = jnp.zeros((16,), jnp.float32)
            return 0

        lax.fori_loop(0, SEG_PER_TILE, zrow, 0)

        def orow(i, _):
            for j in range(D // 16):
                ones[i, pl.ds(j * 16, 16)] = jnp.ones((16,), jnp.float32)
            return 0

        lax.fori_loop(0, SUB, orow, 0)

        seg0 = sid * SEG_PER_TILE
        pltpu.sync_copy(zc, cacc.at[pl.ds(seg0, SEG_PER_TILE)])
        plsc.subcore_barrier()

        base = wid * ROWS_PER_W

        def chunk_body(kk, _):
            row0 = base + kk * CHUNK
            for j in range(NSUB):
                pltpu.sync_copy(b_hbm.at[pl.ds(row0 + j * SUB, SUB)],
                                idxbuf.at[j])
            for j in range(NSUB):
                pltpu.sync_copy(ones, cacc.at[idxbuf.at[j]], add=True)
            return 0

        lax.fori_loop(0, NCHUNK, chunk_body, 0)

        plsc.subcore_barrier()
        pltpu.sync_copy(cacc.at[pl.ds(seg0, SEG_PER_TILE)], zc)
        lane = lax.iota(jnp.int32, 16)

        def crow(g, _):
            res = jnp.zeros((16,), jnp.float32)
            for i in range(16):
                res = jnp.where(lane == i, zc[g * 16 + i, pl.ds(0, 16)], res)
            c1d[pl.ds(g * 16, 16)] = res
            return 0

        lax.fori_loop(0, SEG_PER_TILE // 16, crow, 0)
        pltpu.sync_copy(c1d, cnts_hbm.at[pl.ds(cid * S + seg0, SEG_PER_TILE)])

    return k(batch)


def _tc_finish(sums, cnts, W, b):
    def body(s_ref, c_ref, w_ref, b_ref, o_ref):
        seg = s_ref[0] + s_ref[1]
        cnt = c_ref[0] + c_ref[1]
        pooled = seg / jnp.maximum(cnt, 1.0)
        o_ref[...] = (
            jnp.dot(pooled, w_ref[...].T, preferred_element_type=jnp.float32)
            + b_ref[...]
        )

    return pl.pallas_call(
        body,
        out_shape=jax.ShapeDtypeStruct((S, D), jnp.float32),
    )(sums, cnts.reshape(NC, S, 1), W, b.reshape(1, D))


def kernel(x, batch, data, W, b):
    del data
    batch = batch.astype(jnp.int32)
    sums = _sc_segment_sum(x, batch)
    cnts = _sc_segment_count(batch)
    return _tc_finish(sums, cnts, W, b)


# speedup vs baseline: 5.3571x; 1.2546x over previous
import functools

import jax
import jax.numpy as jnp
from jax import lax
from jax.experimental import pallas as pl
from jax.experimental.pallas import tpu as pltpu
from jax.experimental.pallas import tpu_sc as plsc

N = 320000
D = 128
S = 4096

NC = 2
NS = 16
NW = NC * NS

ROWS_PER_W = N // NW
CHUNK = 400
SUB = 80
NSUB = CHUNK // SUB
NCHUNK = ROWS_PER_W // CHUNK
SEG_PER_TILE = S // NS

_MESH = plsc.VectorSubcoreMesh(core_axis_name="c", subcore_axis_name="s")


def _sc_segment_sum(x, batch):
    @functools.partial(
        pl.kernel,
        out_type=jax.ShapeDtypeStruct((NC, S, D), jnp.float32),
        mesh=_MESH,
        scratch_types=[
            pltpu.VMEM((CHUNK, D), jnp.float32),
            pltpu.VMEM((16, SUB), jnp.int32),
            pltpu.VMEM_SHARED((S, D), jnp.float32),
        ],
    )
    def k(x_hbm, b_hbm, sums_hbm, xbuf, idxbuf, acc):
        cid = lax.axis_index("c")
        sid = lax.axis_index("s")
        wid = cid * NS + sid

        def zrow(i, _):
            for j in range(D // 16):
                xbuf[i, pl.ds(j * 16, 16)] = jnp.zeros((16,), jnp.float32)
            return 0

        lax.fori_loop(0, SEG_PER_TILE, zrow, 0)
        seg0 = sid * SEG_PER_TILE
        pltpu.sync_copy(xbuf.at[pl.ds(0, SEG_PER_TILE)],
                        acc.at[pl.ds(seg0, SEG_PER_TILE)])
        plsc.subcore_barrier()

        base = wid * ROWS_PER_W

        ibase = wid * (ROWS_PER_W // SUB)

        def chunk_body(kk, _):
            row0 = base + kk * CHUNK
            pltpu.sync_copy(x_hbm.at[pl.ds(row0, CHUNK)], xbuf)
            r0 = ibase + kk * NSUB
            a0 = (r0 // 8) * 8
            off = r0 - a0
            pltpu.sync_copy(b_hbm.at[pl.ds(a0, 16)], idxbuf)
            for j in range(NSUB):
                pltpu.sync_copy(xbuf.at[pl.ds(j * SUB, SUB)],
                                acc.at[idxbuf.at[off + j]], add=True)
            return 0

        lax.fori_loop(0, NCHUNK, chunk_body, 0)

        plsc.subcore_barrier()
        pltpu.sync_copy(acc.at[pl.ds(seg0, SEG_PER_TILE)],
                        sums_hbm.at[cid, pl.ds(seg0, SEG_PER_TILE)])

    return k(x, batch)


def _sc_segment_count(batch):
    @functools.partial(
        pl.kernel,
        out_type=jax.ShapeDtypeStruct((NC * S,), jnp.float32),
        mesh=_MESH,
        scratch_types=[
            pltpu.VMEM((16, SUB), jnp.int32),
            pltpu.VMEM((SUB, D), jnp.float32),
            pltpu.VMEM((SEG_PER_TILE, D), jnp.float32),
            pltpu.VMEM((SEG_PER_TILE,), jnp.float32),
            pltpu.VMEM_SHARED((S, D), jnp.float32),
        ],
    )
    def k(b_hbm, cnts_hbm, idxbuf, ones, zc, c1d, cacc):
        cid = lax.axis_index("c")
        sid = lax.axis_index("s")
        wid = cid * NS + sid

        def zrow(i, _):
            for j in range(D // 16):
                zc[i, pl.ds(j * 16, 16)] = jnp.zeros((16,), jnp.float32)
            return 0

        lax.fori_loop(0, SEG_PER_TILE, zrow, 0)

        def orow(i, _):
            for j in range(D // 16):
                ones[i, pl.ds(j * 16, 16)] = jnp.ones((16,), jnp.float32)
            return 0

        lax.fori_loop(0, SUB, orow, 0)

        seg0 = sid * SEG_PER_TILE
        pltpu.sync_copy(zc, cacc.at[pl.ds(seg0, SEG_PER_TILE)])
        plsc.subcore_barrier()

        base = wid * ROWS_PER_W

        ibase = wid * (ROWS_PER_W // SUB)

        def chunk_body(kk, _):
            r0 = ibase + kk * NSUB
            a0 = (r0 // 8) * 8
            off = r0 - a0
            pltpu.sync_copy(b_hbm.at[pl.ds(a0, 16)], idxbuf)
            for j in range(NSUB):
                pltpu.sync_copy(ones, cacc.at[idxbuf.at[off + j]], add=True)
            return 0

        lax.fori_loop(0, NCHUNK, chunk_body, 0)

        plsc.subcore_barrier()
        pltpu.sync_copy(cacc.at[pl.ds(seg0, SEG_PER_TILE)], zc)
        lane = lax.iota(jnp.int32, 16)

        def crow(g, _):
            res = jnp.zeros((16,), jnp.float32)
            for i in range(16):
                res = jnp.where(lane == i, zc[g * 16 + i, pl.ds(0, 16)], res)
            c1d[pl.ds(g * 16, 16)] = res
            return 0

        lax.fori_loop(0, SEG_PER_TILE // 16, crow, 0)
        pltpu.sync_copy(c1d, cnts_hbm.at[pl.ds(cid * S + seg0, SEG_PER_TILE)])

    return k(batch)


def _tc_finish(sums, cnts, W, b):
    def body(s_ref, c_ref, w_ref, b_ref, o_ref):
        seg = s_ref[0] + s_ref[1]
        cnt = c_ref[0] + c_ref[1]
        pooled = seg / jnp.maximum(cnt, 1.0)
        o_ref[...] = (
            jnp.dot(pooled, w_ref[...].T, preferred_element_type=jnp.float32)
            + b_ref[...]
        )

    return pl.pallas_call(
        body,
        out_shape=jax.ShapeDtypeStruct((S, D), jnp.float32),
    )(sums, cnts.reshape(NC, S, 1), W, b.reshape(1, D))


def kernel(x, batch, data, W, b):
    del data
    batch = batch.astype(jnp.int32)
    batch2d = jnp.concatenate(
        [batch.reshape(N // SUB, SUB),
         jnp.zeros((8, SUB), jnp.int32)], axis=0)
    sums = _sc_segment_sum(x, batch2d)
    cnts = _sc_segment_count(batch2d)
    return _tc_finish(sums, cnts, W, b)
